# baseline (speedup 1.0000x reference)
import functools

import jax
import jax.numpy as jnp
from jax import lax
from jax.experimental import pallas as pl
from jax.experimental.pallas import tpu as pltpu
from jax.experimental.pallas import tpu_sc as plsc

_NCHAN, _NPIX, _NNZ = 8, 1024, 2_000_000
_FLAT = _NCHAN * _NPIX * _NPIX
_NC, _NS = 2, 16
_NW = _NC * _NS
_B = 128
_CHUNK = 62_592
_NSTEP = _CHUNK // _B


def _make_gather():
    mesh = plsc.VectorSubcoreMesh(core_axis_name="c", subcore_axis_name="s")

    @functools.partial(
        pl.kernel,
        mesh=mesh,
        out_type=(
            jax.ShapeDtypeStruct((_NNZ,), jnp.float32),
            jax.ShapeDtypeStruct((_NNZ,), jnp.float32),
        ),
        scratch_types=[
            pltpu.VMEM((_B,), jnp.int32),
            pltpu.VMEM((_B,), jnp.float32),
            pltpu.VMEM((_B,), jnp.float32),
            pltpu.SemaphoreType.DMA,
            pltpu.SemaphoreType.DMA,
        ],
    )
    def gather_kernel(re_hbm, im_hbm, idx_hbm, out_re, out_im,
                      idx_v, re_v, im_v, sem_re, sem_im):
        wid = lax.axis_index("s") * _NC + lax.axis_index("c")
        base = jnp.minimum(wid * _CHUNK, _NNZ - _CHUNK)

        def body(j, carry):
            off = base + j * _B
            pltpu.sync_copy(idx_hbm.at[pl.ds(off, _B)], idx_v)
            cp_re = pltpu.async_copy(re_hbm.at[idx_v], re_v, sem_re)
            cp_im = pltpu.async_copy(im_hbm.at[idx_v], im_v, sem_im)
            cp_re.wait()
            cp_im.wait()
            pltpu.sync_copy(re_v, out_re.at[pl.ds(off, _B)])
            pltpu.sync_copy(im_v, out_im.at[pl.ds(off, _B)])
            return carry

        lax.fori_loop(0, _NSTEP, body, 0)

    return gather_kernel


def kernel(modelVisibilityCube_real, modelVisibilityCube_imag, mask_idx):
    re_flat = modelVisibilityCube_real.reshape(-1)
    im_flat = modelVisibilityCube_imag.reshape(-1)
    re, im = _make_gather()(re_flat, im_flat, mask_idx)
    return jax.lax.complex(re, im)


# speedup vs baseline: 2.2699x; 2.2699x over previous
import functools

import jax
import jax.numpy as jnp
from jax import lax
from jax.experimental import pallas as pl
from jax.experimental.pallas import tpu as pltpu
from jax.experimental.pallas import tpu_sc as plsc

_NCHAN, _NPIX, _NNZ = 8, 1024, 2_000_000
_FLAT = _NCHAN * _NPIX * _NPIX
_NC, _NS = 2, 16
_NW = _NC * _NS
_B = 4096
_BLOCKS = 16
_WORK = _B * _BLOCKS


def _make_gather():
    mesh = plsc.VectorSubcoreMesh(core_axis_name="c", subcore_axis_name="s")

    @functools.partial(
        pl.kernel,
        mesh=mesh,
        out_type=(
            jax.ShapeDtypeStruct((_NNZ,), jnp.float32),
            jax.ShapeDtypeStruct((_NNZ,), jnp.float32),
        ),
        scratch_types=[
            pltpu.VMEM((_B,), jnp.int32),
            pltpu.VMEM((_B,), jnp.float32),
            pltpu.VMEM((_B,), jnp.float32),
            pltpu.SemaphoreType.DMA,
            pltpu.SemaphoreType.DMA,
        ],
    )
    def gather_kernel(re_hbm, im_hbm, idx_hbm, out_re, out_im,
                      idx_v, re_v, im_v, sem_re, sem_im):
        wid = lax.axis_index("s") * _NC + lax.axis_index("c")
        base = jnp.minimum(wid * _WORK, _NNZ - _WORK)

        def body(g, carry):
            off = base + g * _B
            pltpu.sync_copy(idx_hbm.at[pl.ds(off, _B)], idx_v)
            cp_re = pltpu.async_copy(re_hbm.at[idx_v], re_v, sem_re)
            cp_im = pltpu.async_copy(im_hbm.at[idx_v], im_v, sem_im)
            cp_re.wait()
            cp_im.wait()
            pltpu.sync_copy(re_v, out_re.at[pl.ds(off, _B)])
            pltpu.sync_copy(im_v, out_im.at[pl.ds(off, _B)])
            return carry

        lax.fori_loop(0, _BLOCKS, body, 0)

    return gather_kernel


def kernel(modelVisibilityCube_real, modelVisibilityCube_imag, mask_idx):
    re_flat = modelVisibilityCube_real.reshape(-1)
    im_flat = modelVisibilityCube_imag.reshape(-1)
    re, im = _make_gather()(re_flat, im_flat, mask_idx)
    return jax.lax.complex(re, im)


# speedup vs baseline: 2.3218x; 1.0229x over previous
import functools

import jax
import jax.numpy as jnp
from jax import lax
from jax.experimental import pallas as pl
from jax.experimental.pallas import tpu as pltpu
from jax.experimental.pallas import tpu_sc as plsc

_NCHAN, _NPIX, _NNZ = 8, 1024, 2_000_000
_NC, _NS = 2, 16
_NW = _NC * _NS
_B = 8192
_NBLK = 8
_WORK = _B * _NBLK


def _make_gather():
    mesh = plsc.VectorSubcoreMesh(core_axis_name="c", subcore_axis_name="s")

    @functools.partial(
        pl.kernel,
        mesh=mesh,
        out_type=(
            jax.ShapeDtypeStruct((_NNZ,), jnp.float32),
            jax.ShapeDtypeStruct((_NNZ,), jnp.float32),
        ),
        scratch_types=[
            pltpu.VMEM((_B,), jnp.int32),
            pltpu.VMEM((_B,), jnp.int32),
            pltpu.VMEM((_B,), jnp.float32),
            pltpu.VMEM((_B,), jnp.float32),
            pltpu.VMEM((_B,), jnp.float32),
            pltpu.VMEM((_B,), jnp.float32),
            pltpu.SemaphoreType.DMA,
            pltpu.SemaphoreType.DMA,
            pltpu.SemaphoreType.DMA,
            pltpu.SemaphoreType.DMA,
        ],
    )
    def gather_kernel(re_hbm, im_hbm, idx_hbm, out_re, out_im,
                      idx0, idx1, re0, re1, im0, im1,
                      sre0, sre1, sim0, sim1):
        idx_b = (idx0, idx1)
        re_b = (re0, re1)
        im_b = (im0, im1)
        sre = (sre0, sre1)
        sim = (sim0, sim1)

        wid = lax.axis_index("s") * _NC + lax.axis_index("c")
        base = jnp.minimum(wid * _WORK, _NNZ - _WORK)

        def fire(g, p):
            pltpu.sync_copy(idx_hbm.at[pl.ds(base + g * _B, _B)], idx_b[p])
            pltpu.async_copy(re_hbm.at[idx_b[p]], re_b[p], sre[p])
            pltpu.async_copy(im_hbm.at[idx_b[p]], im_b[p], sim[p])

        def drain_store(g, p):
            pltpu.make_async_copy(re_hbm.at[idx_b[p]], re_b[p], sre[p]).wait()
            pltpu.make_async_copy(im_hbm.at[idx_b[p]], im_b[p], sim[p]).wait()
            pltpu.sync_copy(re_b[p], out_re.at[pl.ds(base + g * _B, _B)])
            pltpu.sync_copy(im_b[p], out_im.at[pl.ds(base + g * _B, _B)])

        fire(0, 0)

        def body(o, carry):
            for par in range(2):
                g = o * 2 + par

                @pl.when(g + 1 < _NBLK)
                def _():
                    fire(g + 1, 1 - par)

                drain_store(g, par)
            return carry

        lax.fori_loop(0, _NBLK // 2, body, 0)

    return gather_kernel


def kernel(modelVisibilityCube_real, modelVisibilityCube_imag, mask_idx):
    re_flat = modelVisibilityCube_real.reshape(-1)
    im_flat = modelVisibilityCube_imag.reshape(-1)
    re, im = _make_gather()(re_flat, im_flat, mask_idx)
    return jax.lax.complex(re, im)


# speedup vs baseline: 3.1208x; 1.3441x over previous
import functools

import jax
import jax.numpy as jnp
from jax import lax
from jax.experimental import pallas as pl
from jax.experimental.pallas import tpu as pltpu
from jax.experimental.pallas import tpu_sc as plsc

_NCHAN, _NPIX, _NNZ = 8, 1024, 2_000_000
_FLAT = _NCHAN * _NPIX * _NPIX
_NC, _NS = 2, 16
_NW = _NC * _NS
_B = 8192
_NBLK = 8
_WORK = _B * _NBLK
_W = 40960
_L = 16
_UNROLL = 8


def _make_gather():
    mesh = plsc.VectorSubcoreMesh(core_axis_name="c", subcore_axis_name="s")

    @functools.partial(
        pl.kernel,
        mesh=mesh,
        out_type=(
            jax.ShapeDtypeStruct((_NNZ,), jnp.float32),
            jax.ShapeDtypeStruct((_NNZ,), jnp.float32),
        ),
        scratch_types=[
            pltpu.VMEM((_B,), jnp.int32),
            pltpu.VMEM((_B,), jnp.float32),
            pltpu.VMEM((_B,), jnp.float32),
            pltpu.VMEM_SHARED((_NS * _W,), jnp.float32),
            pltpu.VMEM_SHARED((_NS * _W,), jnp.float32),
            pltpu.SemaphoreType.DMA,
            pltpu.SemaphoreType.DMA,
        ],
    )
    def gather_kernel(re_hbm, im_hbm, idx_hbm, out_re, out_im,
                      idx_v, ore_v, oim_v, wre_sh, wim_sh, sem_re, sem_im):
        cid = lax.axis_index("c")
        sid = lax.axis_index("s")
        wid = sid * _NC + cid
        base = jnp.minimum(wid * _WORK, _NNZ - _WORK)
        wbase = sid * _W

        def body(g, carry):
            off = base + g * _B
            pltpu.sync_copy(idx_hbm.at[pl.ds(off, _B)], idx_v)
            lo = idx_v[pl.ds(0, _L)][0]
            hi = idx_v[pl.ds(_B - _L, _L)][_L - 1]
            wstart = jnp.minimum((lo >> 7) << 7, _FLAT - _W)
            fast = (hi - wstart) < _W

            @pl.when(fast)
            def _():
                cw_re = pltpu.async_copy(
                    re_hbm.at[pl.ds(pl.multiple_of(wstart, 128), _W)],
                    wre_sh.at[pl.ds(pl.multiple_of(wbase, 128), _W)], sem_re)
                cw_im = pltpu.async_copy(
                    im_hbm.at[pl.ds(pl.multiple_of(wstart, 128), _W)],
                    wim_sh.at[pl.ds(pl.multiple_of(wbase, 128), _W)], sem_im)

                shift = wstart - wbase

                def inner(o, c):
                    for u in range(_UNROLL):
                        i = (o * _UNROLL + u) * _L
                        idx_v[pl.ds(i, _L)] = idx_v[pl.ds(i, _L)] - shift
                    return c

                lax.fori_loop(0, _B // (_L * _UNROLL), inner, 0)

                cw_re.wait()
                cw_im.wait()
                pltpu.async_copy(wre_sh.at[idx_v], ore_v, sem_re).wait()
                pltpu.async_copy(wim_sh.at[idx_v], oim_v, sem_im).wait()

            @pl.when(jnp.logical_not(fast))
            def _():
                cg_re = pltpu.async_copy(re_hbm.at[idx_v], ore_v, sem_re)
                cg_im = pltpu.async_copy(im_hbm.at[idx_v], oim_v, sem_im)
                cg_re.wait()
                cg_im.wait()

            pltpu.sync_copy(ore_v, out_re.at[pl.ds(off, _B)])
            pltpu.sync_copy(oim_v, out_im.at[pl.ds(off, _B)])
            return carry

        lax.fori_loop(0, _NBLK, body, 0)

    return gather_kernel


def kernel(modelVisibilityCube_real, modelVisibilityCube_imag, mask_idx):
    re_flat = modelVisibilityCube_real.reshape(-1)
    im_flat = modelVisibilityCube_imag.reshape(-1)
    re, im = _make_gather()(re_flat, im_flat, mask_idx)
    return jax.lax.complex(re, im)


# speedup vs baseline: 3.1820x; 1.0196x over previous
import functools

import jax
import jax.numpy as jnp
from jax import lax
from jax.experimental import pallas as pl
from jax.experimental.pallas import tpu as pltpu
from jax.experimental.pallas import tpu_sc as plsc

_NCHAN, _NPIX, _NNZ = 8, 1024, 2_000_000
_FLAT = _NCHAN * _NPIX * _NPIX
_NC, _NS = 2, 16
_NW = _NC * _NS
_B = 4096
_NBLK = 16
_WORK = _B * _NBLK
_W = 24576
_L = 16
_UNROLL = 8


def _make_gather():
    mesh = plsc.VectorSubcoreMesh(core_axis_name="c", subcore_axis_name="s")

    @functools.partial(
        pl.kernel,
        mesh=mesh,
        out_type=(
            jax.ShapeDtypeStruct((_NNZ,), jnp.float32),
            jax.ShapeDtypeStruct((_NNZ,), jnp.float32),
        ),
        scratch_types=[
            pltpu.VMEM((_B,), jnp.int32),
            pltpu.VMEM((_B,), jnp.int32),
            pltpu.VMEM((_B,), jnp.float32),
            pltpu.VMEM((_B,), jnp.float32),
            pltpu.VMEM_SHARED((_NS * _W,), jnp.float32),
            pltpu.VMEM_SHARED((_NS * _W,), jnp.float32),
            pltpu.VMEM_SHARED((_NS * _W,), jnp.float32),
            pltpu.VMEM_SHARED((_NS * _W,), jnp.float32),
            pltpu.SemaphoreType.DMA,
            pltpu.SemaphoreType.DMA,
            pltpu.SemaphoreType.DMA,
            pltpu.SemaphoreType.DMA,
            pltpu.SemaphoreType.DMA,
            pltpu.SemaphoreType.DMA,
            pltpu.SemaphoreType.DMA,
            pltpu.SemaphoreType.DMA,
        ],
    )
    def gather_kernel(re_hbm, im_hbm, idx_hbm, out_re, out_im,
                      idx0, idx1, ore_v, oim_v,
                      wre0, wim0, wre1, wim1,
                      sidx0, sidx1, swre0, swim0, swre1, swim1,
                      sg_re, sg_im):
        idx_b = (idx0, idx1)
        wre_b = (wre0, wre1)
        wim_b = (wim0, wim1)
        sidx = (sidx0, sidx1)
        swre = (swre0, swre1)
        swim = (swim0, swim1)

        cid = lax.axis_index("c")
        sid = lax.axis_index("s")
        wid = sid * _NC + cid
        base = jnp.minimum(wid * _WORK, _NNZ - _WORK)
        wbase = sid * _W
        wbase_dma = pl.multiple_of(wbase, 128)

        def stage(g, p, active):
            iv = idx_b[p]
            lo = iv[pl.ds(0, _L)][0]
            hi = iv[pl.ds(_B - _L, _L)][_L - 1]
            wstart = jnp.minimum((lo >> 7) << 7, _FLAT - _W)
            fast = (hi - wstart) < _W

            @pl.when(fast & active)
            def _():
                pltpu.async_copy(
                    re_hbm.at[pl.ds(pl.multiple_of(wstart, 128), _W)],
                    wre_b[p].at[pl.ds(wbase_dma, _W)], swre[p])
                pltpu.async_copy(
                    im_hbm.at[pl.ds(pl.multiple_of(wstart, 128), _W)],
                    wim_b[p].at[pl.ds(wbase_dma, _W)], swim[p])

                shift = wstart - wbase

                def inner(o, c):
                    for u in range(_UNROLL):
                        i = (o * _UNROLL + u) * _L
                        iv[pl.ds(i, _L)] = iv[pl.ds(i, _L)] - shift
                    return c

                lax.fori_loop(0, _B // (_L * _UNROLL), inner, 0)

            return fast

        def drain(g, p, fast):
            @pl.when(fast)
            def _():
                pltpu.make_async_copy(
                    re_hbm.at[pl.ds(0, _W)],
                    wre_b[p].at[pl.ds(wbase_dma, _W)], swre[p]).wait()
                pltpu.make_async_copy(
                    im_hbm.at[pl.ds(0, _W)],
                    wim_b[p].at[pl.ds(wbase_dma, _W)], swim[p]).wait()
                cg_re = pltpu.async_copy(wre_b[p].at[idx_b[p]], ore_v, sg_re)
                cg_im = pltpu.async_copy(wim_b[p].at[idx_b[p]], oim_v, sg_im)
                cg_re.wait()
                cg_im.wait()

            @pl.when(jnp.logical_not(fast))
            def _():
                cg_re = pltpu.async_copy(re_hbm.at[idx_b[p]], ore_v, sg_re)
                cg_im = pltpu.async_copy(im_hbm.at[idx_b[p]], oim_v, sg_im)
                cg_re.wait()
                cg_im.wait()

            off = base + g * _B
            pltpu.sync_copy(ore_v, out_re.at[pl.ds(off, _B)])
            pltpu.sync_copy(oim_v, out_im.at[pl.ds(off, _B)])

        pltpu.sync_copy(idx_hbm.at[pl.ds(base, _B)], idx0)
        fast0 = stage(0, 0, jnp.bool_(True))
        pltpu.async_copy(idx_hbm.at[pl.ds(base + _B, _B)], idx1, sidx1)

        def body(g, fast_g):
            for par in range(2):
                gg = g * 2 + par
                p = par
                q = 1 - par
                have_next = gg + 1 < _NBLK

                @pl.when(have_next)
                def _():
                    pltpu.make_async_copy(
                        idx_hbm.at[pl.ds(0, _B)], idx_b[q], sidx[q]).wait()

                fast_next = stage(gg + 1, q, have_next)

                drain(gg, p, fast_g)

                @pl.when(gg + 2 < _NBLK)
                def _():
                    pltpu.async_copy(
                        idx_hbm.at[pl.ds(base + (gg + 2) * _B, _B)],
                        idx_b[p], sidx[p])

                fast_g = fast_next
            return fast_g

        lax.fori_loop(0, _NBLK // 2, body, fast0)

    return gather_kernel


def kernel(modelVisibilityCube_real, modelVisibilityCube_imag, mask_idx):
    re_flat = modelVisibilityCube_real.reshape(-1)
    im_flat = modelVisibilityCube_imag.reshape(-1)
    re, im = _make_gather()(re_flat, im_flat, mask_idx)
    return jax.lax.complex(re, im)
